# baseline (speedup 1.0000x reference)
import functools

import jax
import jax.numpy as jnp
from jax import lax
from jax.experimental import pallas as pl
from jax.experimental.pallas import tpu as pltpu
from jax.experimental.pallas import tpu_sc as plsc

_OBS = (100, 100, 100)
_NUM_ACTIONS = 16
_BATCH = 16384

_NC = 2
_NS = 16
_NW = _NC * _NS
_BPW = _BATCH // _NW
_LANES = 16
_GCHUNK = 128
_NGATHER = _BPW // _GCHUNK


@functools.partial(
    pl.kernel,
    out_type=jax.ShapeDtypeStruct((_BATCH, _NUM_ACTIONS), jnp.float32),
    mesh=plsc.VectorSubcoreMesh(core_axis_name="c", subcore_axis_name="s"),
    scratch_types=[
        pltpu.VMEM((_BPW,), jnp.int32),
        pltpu.VMEM((_BPW,), jnp.int32),
        pltpu.VMEM((_BPW,), jnp.int32),
        pltpu.VMEM((_BPW,), jnp.int32),
        pltpu.VMEM((_BPW, _NUM_ACTIONS), jnp.float32),
        pltpu.SemaphoreType.DMA,
    ],
    compiler_params=pltpu.CompilerParams(use_tc_tiling_on_sc=False),
)
def _sc_gather(x0_hbm, x1_hbm, x2_hbm, params_hbm, out_hbm,
               x0_v, x1_v, x2_v, idx_v, rows_v, sem):
    wid = lax.axis_index("s") * _NC + lax.axis_index("c")
    base = wid * _BPW

    pltpu.sync_copy(x0_hbm.at[pl.ds(base, _BPW)], x0_v)
    pltpu.sync_copy(x1_hbm.at[pl.ds(base, _BPW)], x1_v)
    pltpu.sync_copy(x2_hbm.at[pl.ds(base, _BPW)], x2_v)

    hi = jnp.full((_LANES,), _OBS[0] - 1, jnp.int32)
    lo = jnp.zeros((_LANES,), jnp.int32)
    for i in range(_BPW // _LANES):
        sl = pl.ds(i * _LANES, _LANES)
        a = jnp.minimum(jnp.maximum(x0_v[sl], lo), hi)
        b = jnp.minimum(jnp.maximum(x1_v[sl], lo), hi)
        c = jnp.minimum(jnp.maximum(x2_v[sl], lo), hi)
        idx_v[sl] = (a * (_OBS[1] * _OBS[2]) + b * _OBS[2]) + c

    copies = []
    for j in range(_NGATHER):
        sl = pl.ds(j * _GCHUNK, _GCHUNK)
        copies.append(
            pltpu.async_copy(params_hbm.at[idx_v.at[sl]], rows_v.at[sl], sem))
    for cp in copies:
        cp.wait()

    pltpu.sync_copy(rows_v, out_hbm.at[pl.ds(base, _BPW)])


def kernel(x, params):
    x0, x1, x2 = x[0], x[1], x[2]
    return _sc_gather(x0, x1, x2, params)
